# Initial kernel scaffold; baseline (speedup 1.0000x reference)
import jax
import jax.numpy as jnp
from jax.experimental import pallas as pl


def kernel(x, edge_index, W1, b1, W2, b2):
    raise NotImplementedError("write your pallas kernel here")


# speedup vs baseline: 9.1286x; 9.1286x over previous
import functools

import jax
import jax.numpy as jnp
from jax import lax
from jax.experimental import pallas as pl
from jax.experimental.pallas import tpu as pltpu
from jax.experimental.pallas import tpu_sc as plsc

N = 10000
E = 320000
IN_DIM = 128
HID = 32
OUT = 16

CH = 128
NTILES = 32
CHUNKS = -(-E // CH)
CPT = ((-(-CHUNKS // NTILES) + 7) // 8) * 8
CHUNKS_PAD = CPT * NTILES
EPAD = CHUNKS_PAD * CH
OUTN = 10240
DUMP = OUTN
NPAD = OUTN + 8
ROWS_PER_SUBCORE = OUTN // 16


def _make_agg(D):
  mesh = plsc.VectorSubcoreMesh(core_axis_name="c", subcore_axis_name="s")

  @functools.partial(
      pl.kernel,
      out_type=jax.ShapeDtypeStruct((2, OUTN, D), jnp.float32),
      mesh=mesh,
      scratch_types=[
          pltpu.VMEM((CPT, CH), jnp.int32),
          pltpu.VMEM((CPT, CH), jnp.int32),
          pltpu.VMEM((CH, D), jnp.float32),
          pltpu.VMEM_SHARED((NPAD, D), jnp.float32),
          pltpu.SemaphoreType.DMA,
      ],
      compiler_params=pltpu.CompilerParams(use_tc_tiling_on_sc=False),
  )
  def agg(y_hbm, row_hbm, col_hbm, zero_hbm, out_hbm, rowi, coli, gbuf, acc, sem):
    c = lax.axis_index("c")
    s = lax.axis_index("s")
    tile = c * 16 + s

    @pl.when(s == 0)
    def _init():
      pltpu.sync_copy(zero_hbm, acc)

    base = tile * CPT
    pltpu.sync_copy(row_hbm.at[pl.ds(base, CPT)], rowi)
    pltpu.sync_copy(col_hbm.at[pl.ds(base, CPT)], coli)
    plsc.subcore_barrier()

    def body(j, carry):
      pltpu.async_copy(y_hbm.at[coli.at[j]], gbuf, sem).wait()
      pltpu.sync_copy(gbuf, acc.at[rowi.at[j]], add=True)
      return carry

    lax.fori_loop(0, CPT, body, 0)
    plsc.subcore_barrier()
    pltpu.sync_copy(
        acc.at[pl.ds(s * ROWS_PER_SUBCORE, ROWS_PER_SUBCORE)],
        out_hbm.at[c, pl.ds(s * ROWS_PER_SUBCORE, ROWS_PER_SUBCORE)])

  return agg


_agg32 = _make_agg(HID)
_agg16 = _make_agg(OUT)


_RB = 1000


def _mm1_body(x_ref, w_ref, o_ref):
  o_ref[...] = lax.dot_general(
      x_ref[...], w_ref[...], (((1,), (1,)), ((), ())),
      preferred_element_type=jnp.float32)


def _mid_body(p_ref, b_ref, w_ref, o_ref):
  h = jnp.maximum(p_ref[0] + p_ref[1] + b_ref[...], 0.0)
  o_ref[...] = lax.dot_general(
      h, w_ref[...], (((1,), (1,)), ((), ())),
      preferred_element_type=jnp.float32)


def _fin_body(q_ref, b_ref, o_ref):
  o_ref[...] = q_ref[0] + q_ref[1] + b_ref[...]


def kernel(x, edge_index, W1, b1, W2, b2):
  row = edge_index[0].astype(jnp.int32)
  col = edge_index[1].astype(jnp.int32)
  pad = EPAD - E
  row2 = jnp.concatenate([row, jnp.full((pad,), DUMP, jnp.int32)]).reshape(CHUNKS_PAD, CH)
  col2 = jnp.concatenate([col, jnp.zeros((pad,), jnp.int32)]).reshape(CHUNKS_PAD, CH)
  zeros32 = jnp.zeros((NPAD, HID), jnp.float32)
  zeros16 = jnp.zeros((NPAD, OUT), jnp.float32)
  b1r = b1.reshape(1, HID)
  b2r = b2.reshape(1, OUT)

  grid = (N // _RB,)
  y = pl.pallas_call(
      _mm1_body,
      grid=grid,
      in_specs=[pl.BlockSpec((_RB, IN_DIM), lambda i: (i, 0)),
                pl.BlockSpec((HID, IN_DIM), lambda i: (0, 0))],
      out_specs=pl.BlockSpec((_RB, HID), lambda i: (i, 0)),
      out_shape=jax.ShapeDtypeStruct((N, HID), jnp.float32),
  )(x, W1)

  p = _agg32(y, row2, col2, zeros32)

  z = pl.pallas_call(
      _mid_body,
      grid=grid,
      in_specs=[pl.BlockSpec((2, _RB, HID), lambda i: (0, i, 0)),
                pl.BlockSpec((1, HID), lambda i: (0, 0)),
                pl.BlockSpec((OUT, HID), lambda i: (0, 0))],
      out_specs=pl.BlockSpec((_RB, OUT), lambda i: (i, 0)),
      out_shape=jax.ShapeDtypeStruct((N, OUT), jnp.float32),
  )(p, b1r, W2)

  q = _agg16(z, row2, col2, zeros16)

  out = pl.pallas_call(
      _fin_body,
      grid=grid,
      in_specs=[pl.BlockSpec((2, _RB, OUT), lambda i: (0, i, 0)),
                pl.BlockSpec((1, OUT), lambda i: (0, 0))],
      out_specs=pl.BlockSpec((_RB, OUT), lambda i: (i, 0)),
      out_shape=jax.ShapeDtypeStruct((N, OUT), jnp.float32),
  )(q, b2r)

  return out


# speedup vs baseline: 11.7288x; 1.2848x over previous
import functools

import jax
import jax.numpy as jnp
from jax import lax
from jax.experimental import pallas as pl
from jax.experimental.pallas import tpu as pltpu
from jax.experimental.pallas import tpu_sc as plsc

N = 10000
E = 320000
IN_DIM = 128
HID = 32
OUT = 16

CH = 128
NTILES = 32
CHUNKS = -(-E // CH)
CPT = ((-(-CHUNKS // NTILES) + 7) // 8) * 8
CHUNKS_PAD = CPT * NTILES
EPAD = CHUNKS_PAD * CH
OUTN = 10240
DUMP = OUTN
NPAD = OUTN + 8
ROWS_PER_SUBCORE = OUTN // 16


def _make_agg(D):
  mesh = plsc.VectorSubcoreMesh(core_axis_name="c", subcore_axis_name="s")

  @functools.partial(
      pl.kernel,
      out_type=jax.ShapeDtypeStruct((2, OUTN, D), jnp.float32),
      mesh=mesh,
      scratch_types=[
          pltpu.VMEM((CPT, CH), jnp.int32),
          pltpu.VMEM((CPT, CH), jnp.int32),
          pltpu.VMEM((CH, D), jnp.float32),
          pltpu.VMEM((CH, D), jnp.float32),
          pltpu.VMEM_SHARED((NPAD, D), jnp.float32),
          pltpu.SemaphoreType.DMA,
          pltpu.SemaphoreType.DMA,
      ],
      compiler_params=pltpu.CompilerParams(use_tc_tiling_on_sc=False),
  )
  def agg(y_hbm, row_hbm, col_hbm, zero_hbm, out_hbm, rowi, coli, gb0, gb1,
          acc, sem0, sem1):
    c = lax.axis_index("c")
    s = lax.axis_index("s")
    tile = c * 16 + s

    @pl.when(s == 0)
    def _init():
      pltpu.sync_copy(zero_hbm, acc)

    base = tile * CPT
    pltpu.sync_copy(row_hbm.at[pl.ds(base, CPT)], rowi)
    pltpu.sync_copy(col_hbm.at[pl.ds(base, CPT)], coli)
    plsc.subcore_barrier()

    pltpu.async_copy(y_hbm.at[coli.at[0]], gb0, sem0)

    def body(j2, carry):
      j = j2 * 2

      pltpu.async_copy(y_hbm.at[coli.at[j + 1]], gb1, sem1)
      pltpu.make_async_copy(y_hbm.at[coli.at[j]], gb0, sem0).wait()
      pltpu.sync_copy(gb0, acc.at[rowi.at[j]], add=True)

      @pl.when(j + 2 < CPT)
      def _():
        pltpu.async_copy(y_hbm.at[coli.at[j + 2]], gb0, sem0)

      pltpu.make_async_copy(y_hbm.at[coli.at[j + 1]], gb1, sem1).wait()
      pltpu.sync_copy(gb1, acc.at[rowi.at[j + 1]], add=True)
      return carry

    lax.fori_loop(0, CPT // 2, body, 0)
    plsc.subcore_barrier()
    pltpu.sync_copy(
        acc.at[pl.ds(s * ROWS_PER_SUBCORE, ROWS_PER_SUBCORE)],
        out_hbm.at[c, pl.ds(s * ROWS_PER_SUBCORE, ROWS_PER_SUBCORE)])

  return agg


_agg32 = _make_agg(HID)
_agg16 = _make_agg(OUT)


_RB = 1000


def _mm1_body(x_ref, w_ref, o_ref):
  o_ref[...] = lax.dot_general(
      x_ref[...], w_ref[...], (((1,), (1,)), ((), ())),
      preferred_element_type=jnp.float32)


def _mid_body(p_ref, b_ref, w_ref, o_ref):
  h = jnp.maximum(p_ref[0] + p_ref[1] + b_ref[...], 0.0)
  o_ref[...] = lax.dot_general(
      h, w_ref[...], (((1,), (1,)), ((), ())),
      preferred_element_type=jnp.float32)


def _fin_body(q_ref, b_ref, o_ref):
  o_ref[...] = q_ref[0] + q_ref[1] + b_ref[...]


def kernel(x, edge_index, W1, b1, W2, b2):
  row = edge_index[0].astype(jnp.int32)
  col = edge_index[1].astype(jnp.int32)
  pad = EPAD - E
  row2 = jnp.concatenate([row, jnp.full((pad,), DUMP, jnp.int32)]).reshape(CHUNKS_PAD, CH)
  col2 = jnp.concatenate([col, jnp.zeros((pad,), jnp.int32)]).reshape(CHUNKS_PAD, CH)
  zeros32 = jnp.zeros((NPAD, HID), jnp.float32)
  zeros16 = jnp.zeros((NPAD, OUT), jnp.float32)
  b1r = b1.reshape(1, HID)
  b2r = b2.reshape(1, OUT)

  grid = (N // _RB,)
  y = pl.pallas_call(
      _mm1_body,
      grid=grid,
      in_specs=[pl.BlockSpec((_RB, IN_DIM), lambda i: (i, 0)),
                pl.BlockSpec((HID, IN_DIM), lambda i: (0, 0))],
      out_specs=pl.BlockSpec((_RB, HID), lambda i: (i, 0)),
      out_shape=jax.ShapeDtypeStruct((N, HID), jnp.float32),
  )(x, W1)

  p = _agg32(y, row2, col2, zeros32)

  z = pl.pallas_call(
      _mid_body,
      grid=grid,
      in_specs=[pl.BlockSpec((2, _RB, HID), lambda i: (0, i, 0)),
                pl.BlockSpec((1, HID), lambda i: (0, 0)),
                pl.BlockSpec((OUT, HID), lambda i: (0, 0))],
      out_specs=pl.BlockSpec((_RB, OUT), lambda i: (i, 0)),
      out_shape=jax.ShapeDtypeStruct((N, OUT), jnp.float32),
  )(p, b1r, W2)

  q = _agg16(z, row2, col2, zeros16)

  out = pl.pallas_call(
      _fin_body,
      grid=grid,
      in_specs=[pl.BlockSpec((2, _RB, OUT), lambda i: (0, i, 0)),
                pl.BlockSpec((1, OUT), lambda i: (0, 0))],
      out_specs=pl.BlockSpec((_RB, OUT), lambda i: (i, 0)),
      out_shape=jax.ShapeDtypeStruct((N, OUT), jnp.float32),
  )(q, b2r)

  return out


# speedup vs baseline: 18.9345x; 1.6144x over previous
import functools

import jax
import jax.numpy as jnp
from jax import lax
from jax.experimental import pallas as pl
from jax.experimental.pallas import tpu as pltpu
from jax.experimental.pallas import tpu_sc as plsc

N = 10000
E = 320000
IN_DIM = 128
HID = 32
OUT = 16

CH = 128
NTILES = 32
CHUNKS = -(-E // CH)
CPT = ((-(-CHUNKS // NTILES) + 7) // 8) * 8
CHUNKS_PAD = CPT * NTILES
EPAD = CHUNKS_PAD * CH
OUTN = 10240
DUMP = OUTN
NPAD = OUTN + 8
ROWS_PER_SUBCORE = OUTN // 16


def _make_agg(D):
  mesh = plsc.VectorSubcoreMesh(core_axis_name="c", subcore_axis_name="s")

  @functools.partial(
      pl.kernel,
      out_type=jax.ShapeDtypeStruct((2, OUTN, D), jnp.float32),
      mesh=mesh,
      scratch_types=[
          pltpu.VMEM((CPT, CH), jnp.int32),
          pltpu.VMEM((CPT, CH), jnp.int32),
          pltpu.VMEM((CH, D), jnp.float32),
          pltpu.VMEM((CH, D), jnp.float32),
          pltpu.VMEM_SHARED((NPAD, D), jnp.float32),
          pltpu.VMEM_SHARED((N, D), jnp.float32),
          pltpu.SemaphoreType.DMA,
          pltpu.SemaphoreType.DMA,
      ],
      compiler_params=pltpu.CompilerParams(use_tc_tiling_on_sc=False),
  )
  def agg(y_hbm, row_hbm, col_hbm, zero_hbm, out_hbm, rowi, coli, gb0, gb1,
          acc, y_s, sem0, sem1):
    c = lax.axis_index("c")
    s = lax.axis_index("s")
    tile = c * 16 + s

    @pl.when(s == 0)
    def _init():
      pltpu.sync_copy(zero_hbm, acc)

    pltpu.sync_copy(y_hbm.at[pl.ds(s * (N // 16), N // 16)],
                    y_s.at[pl.ds(s * (N // 16), N // 16)])

    base = tile * CPT
    pltpu.sync_copy(row_hbm.at[pl.ds(base, CPT)], rowi)
    pltpu.sync_copy(col_hbm.at[pl.ds(base, CPT)], coli)
    plsc.subcore_barrier()

    pltpu.async_copy(y_s.at[coli.at[0]], gb0, sem0)

    def body(j2, carry):
      j = j2 * 2

      pltpu.async_copy(y_s.at[coli.at[j + 1]], gb1, sem1)
      pltpu.make_async_copy(y_s.at[coli.at[j]], gb0, sem0).wait()
      pltpu.sync_copy(gb0, acc.at[rowi.at[j]], add=True)

      @pl.when(j + 2 < CPT)
      def _():
        pltpu.async_copy(y_s.at[coli.at[j + 2]], gb0, sem0)

      pltpu.make_async_copy(y_s.at[coli.at[j + 1]], gb1, sem1).wait()
      pltpu.sync_copy(gb1, acc.at[rowi.at[j + 1]], add=True)
      return carry

    lax.fori_loop(0, CPT // 2, body, 0)
    plsc.subcore_barrier()
    pltpu.sync_copy(
        acc.at[pl.ds(s * ROWS_PER_SUBCORE, ROWS_PER_SUBCORE)],
        out_hbm.at[c, pl.ds(s * ROWS_PER_SUBCORE, ROWS_PER_SUBCORE)])

  return agg


_agg32 = _make_agg(HID)
_agg16 = _make_agg(OUT)


_RB = 1000


def _mm1_body(x_ref, w_ref, o_ref):
  o_ref[...] = lax.dot_general(
      x_ref[...], w_ref[...], (((1,), (1,)), ((), ())),
      preferred_element_type=jnp.float32)


def _mid_body(p_ref, b_ref, w_ref, o_ref):
  h = jnp.maximum(p_ref[0] + p_ref[1] + b_ref[...], 0.0)
  o_ref[...] = lax.dot_general(
      h, w_ref[...], (((1,), (1,)), ((), ())),
      preferred_element_type=jnp.float32)


def _fin_body(q_ref, b_ref, o_ref):
  o_ref[...] = q_ref[0] + q_ref[1] + b_ref[...]


def kernel(x, edge_index, W1, b1, W2, b2):
  row = edge_index[0].astype(jnp.int32)
  col = edge_index[1].astype(jnp.int32)
  pad = EPAD - E
  row2 = jnp.concatenate([row, jnp.full((pad,), DUMP, jnp.int32)]).reshape(CHUNKS_PAD, CH)
  col2 = jnp.concatenate([col, jnp.zeros((pad,), jnp.int32)]).reshape(CHUNKS_PAD, CH)
  zeros32 = jnp.zeros((NPAD, HID), jnp.float32)
  zeros16 = jnp.zeros((NPAD, OUT), jnp.float32)
  b1r = b1.reshape(1, HID)
  b2r = b2.reshape(1, OUT)

  grid = (N // _RB,)
  y = pl.pallas_call(
      _mm1_body,
      grid=grid,
      in_specs=[pl.BlockSpec((_RB, IN_DIM), lambda i: (i, 0)),
                pl.BlockSpec((HID, IN_DIM), lambda i: (0, 0))],
      out_specs=pl.BlockSpec((_RB, HID), lambda i: (i, 0)),
      out_shape=jax.ShapeDtypeStruct((N, HID), jnp.float32),
  )(x, W1)

  p = _agg32(y, row2, col2, zeros32)

  z = pl.pallas_call(
      _mid_body,
      grid=grid,
      in_specs=[pl.BlockSpec((2, _RB, HID), lambda i: (0, i, 0)),
                pl.BlockSpec((1, HID), lambda i: (0, 0)),
                pl.BlockSpec((OUT, HID), lambda i: (0, 0))],
      out_specs=pl.BlockSpec((_RB, OUT), lambda i: (i, 0)),
      out_shape=jax.ShapeDtypeStruct((N, OUT), jnp.float32),
  )(p, b1r, W2)

  q = _agg16(z, row2, col2, zeros16)

  out = pl.pallas_call(
      _fin_body,
      grid=grid,
      in_specs=[pl.BlockSpec((2, _RB, OUT), lambda i: (0, i, 0)),
                pl.BlockSpec((1, OUT), lambda i: (0, 0))],
      out_specs=pl.BlockSpec((_RB, OUT), lambda i: (i, 0)),
      out_shape=jax.ShapeDtypeStruct((N, OUT), jnp.float32),
  )(q, b2r)

  return out


# speedup vs baseline: 26.2587x; 1.3868x over previous
import functools

import jax
import jax.numpy as jnp
from jax import lax
from jax.experimental import pallas as pl
from jax.experimental.pallas import tpu as pltpu
from jax.experimental.pallas import tpu_sc as plsc

N = 10000
E = 320000
IN_DIM = 128
HID = 32
OUT = 16

CH = 128
NTILES = 32
CHUNKS = E // CH
BASE = CHUNKS // NTILES
NEXTRA = CHUNKS - BASE * NTILES
OUTN = 10240
ROWS_PER_SUBCORE = OUTN // 16
YSTRIPE = N // 16


def _make_agg(D):
  mesh = plsc.VectorSubcoreMesh(core_axis_name="c", subcore_axis_name="s")

  @functools.partial(
      pl.kernel,
      out_type=jax.ShapeDtypeStruct((2, OUTN, D), jnp.float32),
      mesh=mesh,
      scratch_types=[
          pltpu.VMEM((BASE, CH), jnp.int32),
          pltpu.VMEM((BASE, CH), jnp.int32),
          pltpu.VMEM((1, CH), jnp.int32),
          pltpu.VMEM((1, CH), jnp.int32),
          pltpu.VMEM((CH, D), jnp.float32),
          pltpu.VMEM((CH, D), jnp.float32),
          pltpu.VMEM_SHARED((OUTN, D), jnp.float32),
          pltpu.VMEM_SHARED((N, D), jnp.float32),
          pltpu.SemaphoreType.DMA,
          pltpu.SemaphoreType.DMA,
      ],
      compiler_params=pltpu.CompilerParams(use_tc_tiling_on_sc=False),
  )
  def agg(y_hbm, ei3_hbm, zero_hbm, out_hbm, rowi, coli, rowx, colx, gb0, gb1,
          acc, y_s, sem0, sem1):
    c = lax.axis_index("c")
    s = lax.axis_index("s")
    tile = c * 16 + s

    @pl.when(s == 0)
    def _init():
      pltpu.sync_copy(zero_hbm, acc)

    pltpu.sync_copy(y_hbm.at[pl.ds(s * YSTRIPE, YSTRIPE)],
                    y_s.at[pl.ds(s * YSTRIPE, YSTRIPE)])

    start = tile * BASE
    pltpu.sync_copy(ei3_hbm.at[0, pl.ds(start, BASE)], rowi)
    pltpu.sync_copy(ei3_hbm.at[1, pl.ds(start, BASE)], coli)

    @pl.when(tile < NEXTRA)
    def _load_extra():
      pltpu.sync_copy(ei3_hbm.at[0, pl.ds(BASE * NTILES + tile, 1)], rowx)
      pltpu.sync_copy(ei3_hbm.at[1, pl.ds(BASE * NTILES + tile, 1)], colx)

    plsc.subcore_barrier()

    pltpu.async_copy(y_s.at[coli.at[0]], gb0, sem0)

    def body(j2, carry):
      j = j2 * 2

      pltpu.async_copy(y_s.at[coli.at[j + 1]], gb1, sem1)
      pltpu.make_async_copy(y_s.at[coli.at[j]], gb0, sem0).wait()
      pltpu.sync_copy(gb0, acc.at[rowi.at[j]], add=True)

      @pl.when(j + 2 < BASE)
      def _():
        pltpu.async_copy(y_s.at[coli.at[j + 2]], gb0, sem0)

      pltpu.make_async_copy(y_s.at[coli.at[j + 1]], gb1, sem1).wait()
      pltpu.sync_copy(gb1, acc.at[rowi.at[j + 1]], add=True)
      return carry

    lax.fori_loop(0, BASE // 2, body, 0)

    @pl.when(tile < NEXTRA)
    def _do_extra():
      pltpu.async_copy(y_s.at[colx.at[0]], gb0, sem0).wait()
      pltpu.sync_copy(gb0, acc.at[rowx.at[0]], add=True)

    plsc.subcore_barrier()
    pltpu.sync_copy(
        acc.at[pl.ds(s * ROWS_PER_SUBCORE, ROWS_PER_SUBCORE)],
        out_hbm.at[c, pl.ds(s * ROWS_PER_SUBCORE, ROWS_PER_SUBCORE)])

  return agg


_agg32 = _make_agg(HID)
_agg16 = _make_agg(OUT)


_RB1 = 2000
_RB2 = 512


def _mm1_body(x_ref, w_ref, o_ref):
  o_ref[...] = lax.dot_general(
      x_ref[...], w_ref[...], (((1,), (1,)), ((), ())),
      preferred_element_type=jnp.float32)


def _mid_body(p_ref, b_ref, w_ref, o_ref):
  h = jnp.maximum(p_ref[0] + p_ref[1] + b_ref[...], 0.0)
  o_ref[...] = lax.dot_general(
      h, w_ref[...], (((1,), (0,)), ((), ())),
      preferred_element_type=jnp.float32)


def _fin_body(q_ref, b_ref, o_ref):
  o_ref[...] = q_ref[0] + q_ref[1] + b_ref[...]


def kernel(x, edge_index, W1, b1, W2, b2):
  ei3 = edge_index.astype(jnp.int32).reshape(2, CHUNKS, CH)
  zeros32 = jnp.zeros((OUTN, HID), jnp.float32)
  zeros16 = jnp.zeros((OUTN, OUT), jnp.float32)
  w2big = jnp.kron(jnp.eye(4, dtype=jnp.float32), W2.T)
  b1t = jnp.tile(b1, 4).reshape(1, 128)
  b2t = jnp.tile(b2, 8).reshape(1, 128)

  y = pl.pallas_call(
      _mm1_body,
      grid=(N // _RB1,),
      in_specs=[pl.BlockSpec((_RB1, IN_DIM), lambda i: (i, 0)),
                pl.BlockSpec((HID, IN_DIM), lambda i: (0, 0))],
      out_specs=pl.BlockSpec((_RB1, HID), lambda i: (i, 0)),
      out_shape=jax.ShapeDtypeStruct((N, HID), jnp.float32),
  )(x, W1)

  p = _agg32(y, ei3, zeros32)

  p128 = p.reshape(2, OUTN * HID // 128, 128)
  z64 = pl.pallas_call(
      _mid_body,
      grid=(OUTN * HID // 128 // _RB2,),
      in_specs=[pl.BlockSpec((2, _RB2, 128), lambda i: (0, i, 0)),
                pl.BlockSpec((1, 128), lambda i: (0, 0)),
                pl.BlockSpec((128, 64), lambda i: (0, 0))],
      out_specs=pl.BlockSpec((_RB2, 64), lambda i: (i, 0)),
      out_shape=jax.ShapeDtypeStruct((OUTN * HID // 128, 64), jnp.float32),
  )(p128, b1t, w2big)

  z = z64.reshape(OUTN, OUT)
  q = _agg16(z, ei3, zeros16)

  q128 = q.reshape(2, OUTN * OUT // 128, 128)
  o128 = pl.pallas_call(
      _fin_body,
      grid=(1,),
      in_specs=[pl.BlockSpec((2, OUTN * OUT // 128, 128), lambda i: (0, 0, 0)),
                pl.BlockSpec((1, 128), lambda i: (0, 0))],
      out_specs=pl.BlockSpec((OUTN * OUT // 128, 128), lambda i: (i, 0)),
      out_shape=jax.ShapeDtypeStruct((OUTN * OUT // 128, 128), jnp.float32),
  )(q128, b2t)

  return o128[:N * OUT // 128].reshape(N, OUT)
